# Initial kernel scaffold; baseline (speedup 1.0000x reference)
import functools

import jax
import jax.numpy as jnp
from jax import lax
from jax.experimental import pallas as pl
from jax.experimental.pallas import tpu as pltpu
from jax.experimental.pallas import tpu_sc as plsc

N_NODES = 10000
N_EDGES = 320000
D_FEAT = 128

NUM_CORES = 2
NUM_SUBCORES = 16
NUM_TILES = NUM_CORES * NUM_SUBCORES

CHUNK = 80
EDGES_PER_TILE = N_EDGES // NUM_TILES
CHUNKS_PER_TILE = EDGES_PER_TILE // CHUNK
ROWS_PER_SUBCORE = N_NODES // NUM_SUBCORES
ZERO_ROWS = 125


def _transpose_body(x_ref, o_ref):
    o_ref[...] = x_ref[...].T


def _to_node_major(state):
    bn = 2000
    return pl.pallas_call(
        _transpose_body,
        grid=(N_NODES // bn,),
        in_specs=[pl.BlockSpec((D_FEAT, bn), lambda i: (0, i))],
        out_specs=pl.BlockSpec((bn, D_FEAT), lambda i: (i, 0)),
        out_shape=jax.ShapeDtypeStruct((N_NODES, D_FEAT), jnp.float32),
    )(state)


def _combine_body(p_ref, o_ref):
    o_ref[...] = (p_ref[0] + p_ref[1]).T


def _combine(partials):
    bn = 2000
    return pl.pallas_call(
        _combine_body,
        grid=(N_NODES // bn,),
        in_specs=[pl.BlockSpec((2, bn, D_FEAT), lambda i: (0, i, 0))],
        out_specs=pl.BlockSpec((D_FEAT, bn), lambda i: (0, i)),
        out_shape=jax.ShapeDtypeStruct((D_FEAT, N_NODES), jnp.float32),
    )(partials)


def _sc_scatter_add(state_nd, src2d, dst2d):
    mesh = plsc.VectorSubcoreMesh(core_axis_name="c", subcore_axis_name="s")

    @functools.partial(
        pl.kernel,
        out_type=jax.ShapeDtypeStruct((NUM_CORES, N_NODES, D_FEAT), jnp.float32),
        mesh=mesh,
        scratch_types=[
            pltpu.VMEM((CHUNKS_PER_TILE, CHUNK), jnp.int32),
            pltpu.VMEM((CHUNKS_PER_TILE, CHUNK), jnp.int32),
            pltpu.VMEM((CHUNK, D_FEAT), jnp.float32),
            pltpu.VMEM((ZERO_ROWS, D_FEAT), jnp.float32),
            pltpu.VMEM_SHARED((N_NODES, D_FEAT), jnp.float32),
            pltpu.SemaphoreType.DMA,
        ],
    )
    def k(state_hbm, src_hbm, dst_hbm, out_hbm, src_v, dst_v, rows_v, zero_v,
          acc_sh, sem):
        c = lax.axis_index("c")
        s = lax.axis_index("s")
        wid = c * NUM_SUBCORES + s

        row0 = wid * CHUNKS_PER_TILE
        pltpu.sync_copy(src_hbm.at[pl.ds(row0, CHUNKS_PER_TILE)], src_v)
        pltpu.sync_copy(dst_hbm.at[pl.ds(row0, CHUNKS_PER_TILE)], dst_v)

        @pl.loop(0, ZERO_ROWS)
        def _(r):
            @pl.loop(0, D_FEAT, step=16)
            def _(k16):
                zero_v[r, pl.ds(k16, 16)] = jnp.zeros((16,), jnp.float32)

        acc_base = s * ROWS_PER_SUBCORE
        for q in range(ROWS_PER_SUBCORE // ZERO_ROWS):
            pltpu.sync_copy(
                zero_v, acc_sh.at[pl.ds(acc_base + q * ZERO_ROWS, ZERO_ROWS)])
        plsc.subcore_barrier()

        @pl.loop(0, CHUNKS_PER_TILE)
        def _(j):
            pltpu.async_copy(state_hbm.at[src_v.at[j]], rows_v, sem).wait()
            pltpu.sync_copy(rows_v, acc_sh.at[dst_v.at[j]], add=True)

        plsc.subcore_barrier()

        for q in range(ROWS_PER_SUBCORE // ZERO_ROWS):
            r0 = acc_base + q * ZERO_ROWS
            pltpu.sync_copy(acc_sh.at[pl.ds(r0, ZERO_ROWS)],
                            out_hbm.at[c].at[pl.ds(r0, ZERO_ROWS)])

    return k(state_nd, src2d, dst2d)


def kernel(state, edge_index):
    state_nd = _to_node_major(state)
    src2d = edge_index[0].reshape(N_EDGES // CHUNK, CHUNK)
    dst2d = edge_index[1].reshape(N_EDGES // CHUNK, CHUNK)
    partials = _sc_scatter_add(state_nd, src2d, dst2d)
    return _combine(partials)


# speedup vs baseline: 7.2971x; 7.2971x over previous
import functools

import jax
import jax.numpy as jnp
from jax import lax
from jax.experimental import pallas as pl
from jax.experimental.pallas import tpu as pltpu
from jax.experimental.pallas import tpu_sc as plsc

N_NODES = 10000
N_EDGES = 320000
D_FEAT = 128

NUM_CORES = 2
NUM_SUBCORES = 16
NUM_TILES = NUM_CORES * NUM_SUBCORES

CHUNK = 80
EDGES_PER_TILE = N_EDGES // NUM_TILES
CHUNKS_PER_TILE = EDGES_PER_TILE // CHUNK
BLOCK_ROWS = 80
NUM_BLOCKS = N_NODES // BLOCK_ROWS


def _transpose_body(x_ref, o_ref):
    o_ref[...] = x_ref[...].T


def _to_node_major(state):
    return pl.pallas_call(
        _transpose_body,
        out_shape=jax.ShapeDtypeStruct((N_NODES, D_FEAT), jnp.float32),
    )(state)


def _combine_body(p_ref, o_ref):
    o_ref[...] = (p_ref[0] + p_ref[1]).T


def _combine(partials):
    return pl.pallas_call(
        _combine_body,
        out_shape=jax.ShapeDtypeStruct((D_FEAT, N_NODES), jnp.float32),
    )(partials)


def _sc_scatter_add(state_nd, src3d, dst3d):
    mesh = plsc.VectorSubcoreMesh(core_axis_name="c", subcore_axis_name="s")

    @functools.partial(
        pl.kernel,
        out_type=jax.ShapeDtypeStruct((NUM_CORES, N_NODES, D_FEAT), jnp.float32),
        mesh=mesh,
        scratch_types=[
            pltpu.VMEM((CHUNKS_PER_TILE, CHUNK), jnp.int32),
            pltpu.VMEM((CHUNKS_PER_TILE, CHUNK), jnp.int32),
            pltpu.VMEM((CHUNK, D_FEAT), jnp.float32),
            pltpu.VMEM_SHARED((N_NODES, D_FEAT), jnp.float32),
            pltpu.SemaphoreType.DMA,
        ],
    )
    def k(state_hbm, src_hbm, dst_hbm, out_hbm, src_v, dst_v, rows_v,
          acc_sh, sem):
        c = lax.axis_index("c")
        s = lax.axis_index("s")
        wid = c * NUM_SUBCORES + s

        pltpu.sync_copy(src_hbm.at[wid], src_v)
        pltpu.sync_copy(dst_hbm.at[wid], dst_v)

        @pl.loop(0, BLOCK_ROWS)
        def _(r):
            @pl.loop(0, D_FEAT, step=16)
            def _(k16):
                rows_v[r, pl.ds(k16, 16)] = jnp.zeros((16,), jnp.float32)

        @pl.loop(0, NUM_BLOCKS)
        def _(b):
            @pl.when(lax.rem(b, NUM_SUBCORES) == s)
            def _():
                pltpu.sync_copy(
                    rows_v, acc_sh.at[pl.ds(b * BLOCK_ROWS, BLOCK_ROWS)])

        plsc.subcore_barrier()

        @pl.loop(0, CHUNKS_PER_TILE)
        def _(j):
            pltpu.async_copy(state_hbm.at[src_v.at[j]], rows_v, sem).wait()
            pltpu.sync_copy(rows_v, acc_sh.at[dst_v.at[j]], add=True)

        plsc.subcore_barrier()

        @pl.loop(0, NUM_BLOCKS)
        def _(b):
            @pl.when(lax.rem(b, NUM_SUBCORES) == s)
            def _():
                r0 = b * BLOCK_ROWS
                pltpu.sync_copy(acc_sh.at[pl.ds(r0, BLOCK_ROWS)],
                                out_hbm.at[c].at[pl.ds(r0, BLOCK_ROWS)])

    return k(state_nd, src3d, dst3d)


def kernel(state, edge_index):
    state_nd = _to_node_major(state)
    src3d = edge_index[0].reshape(NUM_TILES, CHUNKS_PER_TILE, CHUNK)
    dst3d = edge_index[1].reshape(NUM_TILES, CHUNKS_PER_TILE, CHUNK)
    partials = _sc_scatter_add(state_nd, src3d, dst3d)
    return _combine(partials)


# speedup vs baseline: 9.2811x; 1.2719x over previous
import functools

import jax
import jax.numpy as jnp
from jax import lax
from jax.experimental import pallas as pl
from jax.experimental.pallas import tpu as pltpu
from jax.experimental.pallas import tpu_sc as plsc

N_NODES = 10000
N_EDGES = 320000
D_FEAT = 128

NUM_CORES = 2
NUM_SUBCORES = 16
NUM_TILES = NUM_CORES * NUM_SUBCORES

CHUNK = 80
EDGES_PER_TILE = N_EDGES // NUM_TILES
CHUNKS_PER_TILE = EDGES_PER_TILE // CHUNK
NBUF = 2
LOOKAHEAD = 1
MAIN_CHUNKS = (CHUNKS_PER_TILE // NBUF) * NBUF
BLOCK_ROWS = 80
NUM_BLOCKS = N_NODES // BLOCK_ROWS


def _transpose_body(x_ref, o_ref):
    o_ref[...] = x_ref[...].T


def _to_node_major(state):
    return pl.pallas_call(
        _transpose_body,
        out_shape=jax.ShapeDtypeStruct((N_NODES, D_FEAT), jnp.float32),
    )(state)


def _combine_body(p_ref, o_ref):
    o_ref[...] = (p_ref[0] + p_ref[1]).T


def _combine(partials):
    return pl.pallas_call(
        _combine_body,
        out_shape=jax.ShapeDtypeStruct((D_FEAT, N_NODES), jnp.float32),
    )(partials)


def _sc_scatter_add(state_nd, src1d, dst1d):
    mesh = plsc.VectorSubcoreMesh(core_axis_name="c", subcore_axis_name="s")

    @functools.partial(
        pl.kernel,
        out_type=jax.ShapeDtypeStruct((NUM_CORES, N_NODES, D_FEAT), jnp.float32),
        mesh=mesh,
        scratch_types=[
            pltpu.VMEM((EDGES_PER_TILE,), jnp.int32),
            pltpu.VMEM((EDGES_PER_TILE,), jnp.int32),
        ] + [
            pltpu.VMEM((CHUNK, D_FEAT), jnp.float32) for _ in range(NBUF)
        ] + [
            pltpu.VMEM_SHARED((N_NODES, D_FEAT), jnp.float32),
        ] + [pltpu.SemaphoreType.DMA for _ in range(2 * NBUF)],
    )
    def k(state_hbm, src_hbm, dst_hbm, out_hbm, src_v, dst_v, *rest):
        rows = rest[:NBUF]
        acc_sh = rest[NBUF]
        gsem = rest[NBUF + 1:2 * NBUF + 1]
        ssem = rest[2 * NBUF + 1:3 * NBUF + 1]
        c = lax.axis_index("c")
        s = lax.axis_index("s")
        wid = c * NUM_SUBCORES + s

        def gather_start(jj, b):
            pltpu.async_copy(
                state_hbm.at[src_v.at[pl.ds(jj * CHUNK, CHUNK)]],
                rows[b], gsem[b])

        def gather_wait(jj, b):
            pltpu.make_async_copy(
                state_hbm.at[src_v.at[pl.ds(jj * CHUNK, CHUNK)]],
                rows[b], gsem[b]).wait()

        def scatter_start(jj, b):
            pltpu.async_copy(
                rows[b], acc_sh.at[dst_v.at[pl.ds(jj * CHUNK, CHUNK)]],
                ssem[b], add=True)

        def scatter_wait(jj, b):
            pltpu.make_async_copy(
                rows[b], acc_sh.at[dst_v.at[pl.ds(jj * CHUNK, CHUNK)]],
                ssem[b]).wait()

        e0 = wid * EDGES_PER_TILE
        pltpu.sync_copy(src_hbm.at[pl.ds(e0, EDGES_PER_TILE)], src_v)
        pltpu.sync_copy(dst_hbm.at[pl.ds(e0, EDGES_PER_TILE)], dst_v)

        @pl.loop(0, BLOCK_ROWS)
        def _(r):
            @pl.loop(0, D_FEAT, step=16)
            def _(k16):
                rows[0][r, pl.ds(k16, 16)] = jnp.zeros((16,), jnp.float32)

        @pl.loop(0, NUM_BLOCKS)
        def _(b):
            @pl.when(lax.rem(b, NUM_SUBCORES) == s)
            def _():
                pltpu.sync_copy(
                    rows[0], acc_sh.at[pl.ds(b * BLOCK_ROWS, BLOCK_ROWS)])

        plsc.subcore_barrier()

        for b in range(LOOKAHEAD):
            gather_start(b, b)

        @pl.loop(0, MAIN_CHUNKS, step=NBUF)
        def _(j):
            for b in range(NBUF):
                jj = j + b
                gather_wait(jj, b)
                scatter_start(jj, b)
                bf = (b + LOOKAHEAD) % NBUF
                f = jj + LOOKAHEAD

                @pl.when(f < CHUNKS_PER_TILE)
                def _():
                    @pl.when(f >= NBUF)
                    def _():
                        scatter_wait(f - NBUF, bf)

                    gather_start(f, bf)

        for jj in range(MAIN_CHUNKS, CHUNKS_PER_TILE):
            b = jj % NBUF
            gather_wait(jj, b)
            scatter_start(jj, b)

        for jj in range(CHUNKS_PER_TILE - NBUF, CHUNKS_PER_TILE):
            scatter_wait(jj, jj % NBUF)

        plsc.subcore_barrier()

        @pl.loop(0, NUM_BLOCKS)
        def _(b):
            @pl.when(lax.rem(b, NUM_SUBCORES) == s)
            def _():
                r0 = b * BLOCK_ROWS
                pltpu.sync_copy(acc_sh.at[pl.ds(r0, BLOCK_ROWS)],
                                out_hbm.at[c].at[pl.ds(r0, BLOCK_ROWS)])

    return k(state_nd, src1d, dst1d)


def kernel(state, edge_index):
    state_nd = _to_node_major(state)
    partials = _sc_scatter_add(state_nd, edge_index[0], edge_index[1])
    return _combine(partials)


# speedup vs baseline: 9.8699x; 1.0634x over previous
import functools

import jax
import jax.numpy as jnp
from jax import lax
from jax.experimental import pallas as pl
from jax.experimental.pallas import tpu as pltpu
from jax.experimental.pallas import tpu_sc as plsc

N_NODES = 10000
N_PAD = 10240
N_EDGES = 320000
D_FEAT = 128

NUM_CORES = 2
NUM_SUBCORES = 16
NUM_TILES = NUM_CORES * NUM_SUBCORES

CHUNK = 120
EDGES_PER_TILE = N_EDGES // NUM_TILES
EDGES_PER_TILE_PAD = -(-EDGES_PER_TILE // CHUNK) * CHUNK
CHUNKS_PER_TILE = EDGES_PER_TILE_PAD // CHUNK
NBUF = 2
LOOKAHEAD = 1
MAIN_CHUNKS = (CHUNKS_PER_TILE // NBUF) * NBUF
BLOCK_ROWS = 80
NUM_BLOCKS = N_NODES // BLOCK_ROWS

TC_GRID = 8
BN = N_PAD // TC_GRID


def _transpose_body(x_ref, o_ref):
    o_ref[...] = x_ref[...].T


def _to_node_major(state_pad):
    return pl.pallas_call(
        _transpose_body,
        grid=(TC_GRID,),
        in_specs=[pl.BlockSpec((D_FEAT, BN), lambda i: (0, i))],
        out_specs=pl.BlockSpec((BN, D_FEAT), lambda i: (i, 0)),
        out_shape=jax.ShapeDtypeStruct((N_PAD, D_FEAT), jnp.float32),
    )(state_pad)


def _combine_body(p_ref, o_ref):
    o_ref[...] = (p_ref[0] + p_ref[1]).T


def _combine(partials):
    return pl.pallas_call(
        _combine_body,
        grid=(TC_GRID,),
        in_specs=[pl.BlockSpec((2, BN, D_FEAT), lambda i: (0, i, 0))],
        out_specs=pl.BlockSpec((D_FEAT, BN), lambda i: (0, i)),
        out_shape=jax.ShapeDtypeStruct((D_FEAT, N_PAD), jnp.float32),
    )(partials)


def _sc_scatter_add(state_nd, src1d, dst1d):
    mesh = plsc.VectorSubcoreMesh(core_axis_name="c", subcore_axis_name="s")

    @functools.partial(
        pl.kernel,
        out_type=jax.ShapeDtypeStruct((NUM_CORES, N_PAD, D_FEAT), jnp.float32),
        mesh=mesh,
        scratch_types=[
            pltpu.VMEM((EDGES_PER_TILE_PAD,), jnp.int32),
            pltpu.VMEM((EDGES_PER_TILE_PAD,), jnp.int32),
        ] + [
            pltpu.VMEM((CHUNK, D_FEAT), jnp.float32) for _ in range(NBUF)
        ] + [
            pltpu.VMEM_SHARED((N_NODES, D_FEAT), jnp.float32),
        ] + [pltpu.SemaphoreType.DMA for _ in range(2 * NBUF)],
    )
    def k(state_hbm, src_hbm, dst_hbm, out_hbm, src_v, dst_v, *rest):
        rows = rest[:NBUF]
        acc_sh = rest[NBUF]
        gsem = rest[NBUF + 1:2 * NBUF + 1]
        ssem = rest[2 * NBUF + 1:3 * NBUF + 1]
        c = lax.axis_index("c")
        s = lax.axis_index("s")
        wid = c * NUM_SUBCORES + s

        def gather_start(jj, b):
            pltpu.async_copy(
                state_hbm.at[src_v.at[pl.ds(jj * CHUNK, CHUNK)]],
                rows[b], gsem[b])

        def gather_wait(jj, b):
            pltpu.make_async_copy(
                state_hbm.at[src_v.at[pl.ds(jj * CHUNK, CHUNK)]],
                rows[b], gsem[b]).wait()

        def scatter_start(jj, b):
            pltpu.async_copy(
                rows[b], acc_sh.at[dst_v.at[pl.ds(jj * CHUNK, CHUNK)]],
                ssem[b], add=True)

        def scatter_wait(jj, b):
            pltpu.make_async_copy(
                rows[b], acc_sh.at[dst_v.at[pl.ds(jj * CHUNK, CHUNK)]],
                ssem[b]).wait()

        e0 = wid * EDGES_PER_TILE_PAD
        pltpu.sync_copy(src_hbm.at[pl.ds(e0, EDGES_PER_TILE_PAD)], src_v)
        pltpu.sync_copy(dst_hbm.at[pl.ds(e0, EDGES_PER_TILE_PAD)], dst_v)

        @pl.loop(0, BLOCK_ROWS)
        def _(r):
            @pl.loop(0, D_FEAT, step=16)
            def _(k16):
                rows[0][r, pl.ds(k16, 16)] = jnp.zeros((16,), jnp.float32)

        @pl.loop(0, NUM_BLOCKS)
        def _(b):
            @pl.when(lax.rem(b, NUM_SUBCORES) == s)
            def _():
                pltpu.sync_copy(
                    rows[0].at[pl.ds(0, BLOCK_ROWS)],
                    acc_sh.at[pl.ds(b * BLOCK_ROWS, BLOCK_ROWS)])

        plsc.subcore_barrier()

        for b in range(LOOKAHEAD):
            gather_start(b, b)

        @pl.loop(0, MAIN_CHUNKS, step=NBUF)
        def _(j):
            for b in range(NBUF):
                jj = j + b
                gather_wait(jj, b)
                scatter_start(jj, b)
                bf = (b + LOOKAHEAD) % NBUF
                f = jj + LOOKAHEAD

                @pl.when(f < CHUNKS_PER_TILE)
                def _():
                    @pl.when(f >= NBUF)
                    def _():
                        scatter_wait(f - NBUF, bf)

                    gather_start(f, bf)

        for jj in range(MAIN_CHUNKS, CHUNKS_PER_TILE):
            b = jj % NBUF
            gather_wait(jj, b)
            scatter_start(jj, b)

        for jj in range(CHUNKS_PER_TILE - NBUF, CHUNKS_PER_TILE):
            scatter_wait(jj, jj % NBUF)

        plsc.subcore_barrier()

        @pl.loop(0, NUM_BLOCKS)
        def _(b):
            @pl.when(lax.rem(b, NUM_SUBCORES) == s)
            def _():
                r0 = b * BLOCK_ROWS
                pltpu.sync_copy(acc_sh.at[pl.ds(r0, BLOCK_ROWS)],
                                out_hbm.at[c].at[pl.ds(r0, BLOCK_ROWS)])

    return k(state_nd, src1d, dst1d)


def kernel(state, edge_index):
    state_pad = jnp.pad(state, ((0, 0), (0, N_PAD - N_NODES)))
    state_nd = _to_node_major(state_pad)

    n_pad_edges = EDGES_PER_TILE_PAD - EDGES_PER_TILE
    pad_ids = jnp.arange(NUM_TILES * n_pad_edges, dtype=jnp.int32)
    pad_src = (N_NODES + pad_ids % (N_PAD - N_NODES)).reshape(
        NUM_TILES, n_pad_edges)
    pad_dst = (pad_ids * 37 % N_NODES).reshape(NUM_TILES, n_pad_edges)
    src1d = jnp.concatenate(
        [edge_index[0].reshape(NUM_TILES, EDGES_PER_TILE), pad_src],
        axis=1).reshape(-1)
    dst1d = jnp.concatenate(
        [edge_index[1].reshape(NUM_TILES, EDGES_PER_TILE), pad_dst],
        axis=1).reshape(-1)

    partials = _sc_scatter_add(state_nd, src1d, dst1d)
    return _combine(partials)[:, :N_NODES]


# speedup vs baseline: 10.6562x; 1.0797x over previous
import functools

import jax
import jax.numpy as jnp
from jax import lax
from jax.experimental import pallas as pl
from jax.experimental.pallas import tpu as pltpu
from jax.experimental.pallas import tpu_sc as plsc

N_NODES = 10000
N_EDGES = 320000
D_FEAT = 128

NUM_CORES = 2
NUM_SUBCORES = 16
NUM_TILES = NUM_CORES * NUM_SUBCORES

CHUNK = 120
EDGES_PER_TILE = N_EDGES // NUM_TILES
FULL_CHUNKS = EDGES_PER_TILE // CHUNK
TAIL = EDGES_PER_TILE - FULL_CHUNKS * CHUNK
TOTAL_CHUNKS = FULL_CHUNKS + (1 if TAIL else 0)
NBUF = 2
LOOKAHEAD = 1
MAIN_CHUNKS = ((TOTAL_CHUNKS - 2) // NBUF) * NBUF
BLOCK_ROWS = 80
NUM_BLOCKS = N_NODES // BLOCK_ROWS


def _chunk_size(jj):
    return CHUNK if jj < FULL_CHUNKS else TAIL


def _transpose_body(x_ref, o_ref):
    o_ref[...] = x_ref[...].T


def _to_node_major(state):
    return pl.pallas_call(
        _transpose_body,
        out_shape=jax.ShapeDtypeStruct((N_NODES, D_FEAT), jnp.float32),
    )(state)


def _combine_body(p_ref, o_ref):
    o_ref[...] = (p_ref[0] + p_ref[1]).T


def _combine(partials):
    return pl.pallas_call(
        _combine_body,
        out_shape=jax.ShapeDtypeStruct((D_FEAT, N_NODES), jnp.float32),
    )(partials)


def _sc_scatter_add(state_nd, src1d, dst1d):
    mesh = plsc.VectorSubcoreMesh(core_axis_name="c", subcore_axis_name="s")

    @functools.partial(
        pl.kernel,
        out_type=jax.ShapeDtypeStruct((NUM_CORES, N_NODES, D_FEAT), jnp.float32),
        mesh=mesh,
        scratch_types=[
            pltpu.VMEM((EDGES_PER_TILE,), jnp.int32),
            pltpu.VMEM((EDGES_PER_TILE,), jnp.int32),
        ] + [
            pltpu.VMEM((CHUNK, D_FEAT), jnp.float32) for _ in range(NBUF)
        ] + [
            pltpu.VMEM_SHARED((N_NODES, D_FEAT), jnp.float32),
        ] + [pltpu.SemaphoreType.DMA for _ in range(2 * NBUF + 2)],
    )
    def k(state_hbm, src_hbm, dst_hbm, out_hbm, src_v, dst_v, *rest):
        rows = rest[:NBUF]
        acc_sh = rest[NBUF]
        gsem = rest[NBUF + 1:2 * NBUF + 1]
        ssem = rest[2 * NBUF + 1:3 * NBUF + 1]
        isem = rest[3 * NBUF + 1:3 * NBUF + 3]
        c = lax.axis_index("c")
        s = lax.axis_index("s")
        wid = c * NUM_SUBCORES + s

        def buf(b, n):
            return rows[b] if n == CHUNK else rows[b].at[pl.ds(0, n)]

        def gather_start(jj, b, n=CHUNK):
            pltpu.async_copy(
                state_hbm.at[src_v.at[pl.ds(jj * CHUNK, n)]],
                buf(b, n), gsem[b])

        def gather_wait(jj, b, n=CHUNK):
            pltpu.make_async_copy(
                state_hbm.at[src_v.at[pl.ds(jj * CHUNK, n)]],
                buf(b, n), gsem[b]).wait()

        def scatter_start(jj, b, n=CHUNK):
            pltpu.async_copy(
                buf(b, n), acc_sh.at[dst_v.at[pl.ds(jj * CHUNK, n)]],
                ssem[b], add=True)

        def scatter_wait(jj, b, n=CHUNK):
            pltpu.make_async_copy(
                buf(b, n), acc_sh.at[dst_v.at[pl.ds(jj * CHUNK, n)]],
                ssem[b]).wait()

        e0 = wid * EDGES_PER_TILE
        idx_src = pltpu.async_copy(
            src_hbm.at[pl.ds(e0, EDGES_PER_TILE)], src_v, isem[0])
        idx_dst = pltpu.async_copy(
            dst_hbm.at[pl.ds(e0, EDGES_PER_TILE)], dst_v, isem[1])

        @pl.loop(0, BLOCK_ROWS)
        def _(r):
            @pl.loop(0, D_FEAT, step=16)
            def _(k16):
                rows[0][r, pl.ds(k16, 16)] = jnp.zeros((16,), jnp.float32)

        @pl.loop(0, NUM_BLOCKS)
        def _(b):
            @pl.when(lax.rem(b, NUM_SUBCORES) == s)
            def _():
                pltpu.sync_copy(
                    rows[0].at[pl.ds(0, BLOCK_ROWS)],
                    acc_sh.at[pl.ds(b * BLOCK_ROWS, BLOCK_ROWS)])

        idx_src.wait()
        idx_dst.wait()
        plsc.subcore_barrier()

        for b in range(LOOKAHEAD):
            gather_start(b, b)

        @pl.loop(0, MAIN_CHUNKS, step=NBUF)
        def _(j):
            for b in range(NBUF):
                jj = j + b
                gather_wait(jj, b)
                scatter_start(jj, b)
                bf = (b + LOOKAHEAD) % NBUF
                f = jj + LOOKAHEAD

                @pl.when(f >= NBUF)
                def _():
                    scatter_wait(f - NBUF, bf)

                gather_start(f, bf)

        for jj in range(MAIN_CHUNKS, TOTAL_CHUNKS):
            b = jj % NBUF
            gather_wait(jj, b, _chunk_size(jj))
            scatter_start(jj, b, _chunk_size(jj))
            f = jj + LOOKAHEAD
            if f < TOTAL_CHUNKS:
                bf = f % NBUF
                scatter_wait(f - NBUF, bf, _chunk_size(f - NBUF))
                gather_start(f, bf, _chunk_size(f))

        for jj in range(TOTAL_CHUNKS - NBUF, TOTAL_CHUNKS):
            scatter_wait(jj, jj % NBUF, _chunk_size(jj))

        plsc.subcore_barrier()

        @pl.loop(0, NUM_BLOCKS)
        def _(b):
            @pl.when(lax.rem(b, NUM_SUBCORES) == s)
            def _():
                r0 = b * BLOCK_ROWS
                pltpu.sync_copy(acc_sh.at[pl.ds(r0, BLOCK_ROWS)],
                                out_hbm.at[c].at[pl.ds(r0, BLOCK_ROWS)])

    return k(state_nd, src1d, dst1d)


def kernel(state, edge_index):
    state_nd = _to_node_major(state)
    partials = _sc_scatter_add(state_nd, edge_index[0], edge_index[1])
    return _combine(partials)


# speedup vs baseline: 12.9001x; 1.2106x over previous
import functools

import jax
import jax.numpy as jnp
from jax import lax
from jax.experimental import pallas as pl
from jax.experimental.pallas import tpu as pltpu
from jax.experimental.pallas import tpu_sc as plsc

N_NODES = 10000
N_EDGES = 320000
D_FEAT = 128

NUM_CORES = 2
NUM_SUBCORES = 16
NUM_TILES = NUM_CORES * NUM_SUBCORES

CHUNK = 80
EDGES_PER_TILE = N_EDGES // NUM_TILES
FULL_CHUNKS = EDGES_PER_TILE // CHUNK
TAIL = EDGES_PER_TILE - FULL_CHUNKS * CHUNK
TOTAL_CHUNKS = FULL_CHUNKS + (1 if TAIL else 0)
NBUF = 3
LOOKAHEAD = 2
MAIN_CHUNKS = ((TOTAL_CHUNKS - LOOKAHEAD) // NBUF) * NBUF
BLOCK_ROWS = 80
NUM_BLOCKS = N_NODES // BLOCK_ROWS


def _chunk_size(jj):
    return CHUNK if jj < FULL_CHUNKS else TAIL


def _transpose_body(x_ref, o_ref):
    o_ref[...] = x_ref[...].T


def _to_node_major(state):
    return pl.pallas_call(
        _transpose_body,
        out_shape=jax.ShapeDtypeStruct((N_NODES, D_FEAT), jnp.float32),
    )(state)


def _combine_body(p_ref, o_ref):
    o_ref[...] = (p_ref[0] + p_ref[1]).T


def _combine(partials):
    return pl.pallas_call(
        _combine_body,
        out_shape=jax.ShapeDtypeStruct((D_FEAT, N_NODES), jnp.float32),
    )(partials)


def _sc_scatter_add(state_nd, src1d, dst1d):
    mesh = plsc.VectorSubcoreMesh(core_axis_name="c", subcore_axis_name="s")

    @functools.partial(
        pl.kernel,
        out_type=jax.ShapeDtypeStruct((NUM_CORES, N_NODES, D_FEAT), jnp.float32),
        mesh=mesh,
        scratch_types=[
            pltpu.VMEM((EDGES_PER_TILE,), jnp.int32),
            pltpu.VMEM((EDGES_PER_TILE,), jnp.int32),
        ] + [
            pltpu.VMEM((CHUNK, D_FEAT), jnp.float32) for _ in range(NBUF)
        ] + [
            pltpu.VMEM_SHARED((N_NODES, D_FEAT), jnp.float32),
        ] + [pltpu.SemaphoreType.DMA for _ in range(2 * NBUF + 2)],
    )
    def k(state_hbm, src_hbm, dst_hbm, out_hbm, src_v, dst_v, *rest):
        rows = rest[:NBUF]
        acc_sh = rest[NBUF]
        gsem = rest[NBUF + 1:2 * NBUF + 1]
        ssem = rest[2 * NBUF + 1:3 * NBUF + 1]
        isem = rest[3 * NBUF + 1:3 * NBUF + 3]
        c = lax.axis_index("c")
        s = lax.axis_index("s")
        wid = c * NUM_SUBCORES + s

        def buf(b, n):
            return rows[b] if n == CHUNK else rows[b].at[pl.ds(0, n)]

        def gather_start(jj, b, n=CHUNK):
            pltpu.async_copy(
                state_hbm.at[src_v.at[pl.ds(jj * CHUNK, n)]],
                buf(b, n), gsem[b])

        def gather_wait(jj, b, n=CHUNK):
            pltpu.make_async_copy(
                state_hbm.at[src_v.at[pl.ds(jj * CHUNK, n)]],
                buf(b, n), gsem[b]).wait()

        def scatter_start(jj, b, n=CHUNK):
            pltpu.async_copy(
                buf(b, n), acc_sh.at[dst_v.at[pl.ds(jj * CHUNK, n)]],
                ssem[b], add=True)

        def scatter_wait(jj, b, n=CHUNK):
            pltpu.make_async_copy(
                buf(b, n), acc_sh.at[dst_v.at[pl.ds(jj * CHUNK, n)]],
                ssem[b]).wait()

        e0 = wid * EDGES_PER_TILE
        idx_src = pltpu.async_copy(
            src_hbm.at[pl.ds(e0, EDGES_PER_TILE)], src_v, isem[0])
        idx_dst = pltpu.async_copy(
            dst_hbm.at[pl.ds(e0, EDGES_PER_TILE)], dst_v, isem[1])

        @pl.loop(0, BLOCK_ROWS)
        def _(r):
            @pl.loop(0, D_FEAT, step=16)
            def _(k16):
                rows[0][r, pl.ds(k16, 16)] = jnp.zeros((16,), jnp.float32)

        @pl.loop(0, NUM_BLOCKS)
        def _(b):
            @pl.when(lax.rem(b, NUM_SUBCORES) == s)
            def _():
                pltpu.sync_copy(
                    rows[0].at[pl.ds(0, BLOCK_ROWS)],
                    acc_sh.at[pl.ds(b * BLOCK_ROWS, BLOCK_ROWS)])

        idx_src.wait()
        idx_dst.wait()
        plsc.subcore_barrier()

        for b in range(LOOKAHEAD):
            gather_start(b, b)

        @pl.loop(0, MAIN_CHUNKS, step=NBUF)
        def _(j):
            for b in range(NBUF):
                jj = j + b
                gather_wait(jj, b)
                scatter_start(jj, b)
                bf = (b + LOOKAHEAD) % NBUF
                f = jj + LOOKAHEAD

                @pl.when(f >= NBUF)
                def _():
                    scatter_wait(f - NBUF, bf)

                gather_start(f, bf)

        for jj in range(MAIN_CHUNKS, TOTAL_CHUNKS):
            b = jj % NBUF
            gather_wait(jj, b, _chunk_size(jj))
            scatter_start(jj, b, _chunk_size(jj))
            f = jj + LOOKAHEAD
            if f < TOTAL_CHUNKS:
                bf = f % NBUF
                scatter_wait(f - NBUF, bf, _chunk_size(f - NBUF))
                gather_start(f, bf, _chunk_size(f))

        for jj in range(TOTAL_CHUNKS - NBUF, TOTAL_CHUNKS):
            scatter_wait(jj, jj % NBUF, _chunk_size(jj))

        plsc.subcore_barrier()

        @pl.loop(0, NUM_BLOCKS)
        def _(b):
            @pl.when(lax.rem(b, NUM_SUBCORES) == s)
            def _():
                r0 = b * BLOCK_ROWS
                pltpu.sync_copy(acc_sh.at[pl.ds(r0, BLOCK_ROWS)],
                                out_hbm.at[c].at[pl.ds(r0, BLOCK_ROWS)])

    return k(state_nd, src1d, dst1d)


def kernel(state, edge_index):
    state_nd = _to_node_major(state)
    partials = _sc_scatter_add(state_nd, edge_index[0], edge_index[1])
    return _combine(partials)
